# baseline (speedup 1.0000x reference)
import functools

import jax
import jax.numpy as jnp
from jax import lax
from jax.experimental import pallas as pl
from jax.experimental.pallas import tpu as pltpu
from jax.experimental.pallas import tpu_sc as plsc


def _make_min(n_out):
    mesh = plsc.VectorSubcoreMesh(core_axis_name="c", subcore_axis_name="s")

    @functools.partial(
        pl.kernel,
        out_type=jax.ShapeDtypeStruct((n_out,), jnp.float32),
        mesh=mesh,
        scratch_types=[pltpu.VMEM((16,), jnp.float32)],
        compiler_params=pltpu.CompilerParams(needs_layout_passes=False),
    )
    def sc_min(out_hbm, buf_v):
        wid = lax.axis_index("s") * 2 + lax.axis_index("c")

        @pl.when(wid == 0)
        def _():
            pltpu.sync_copy(buf_v, out_hbm.at[pl.ds(0, 16)])

    return sc_min


def kernel(idx, outputs):
    b, t = idx.shape
    out_flat = _make_min(b * t * 3)()
    return out_flat.reshape(b, t, 3)


# speedup vs baseline: 3.1331x; 3.1331x over previous
import functools

import jax
import jax.numpy as jnp
from jax import lax
from jax.experimental import pallas as pl
from jax.experimental.pallas import tpu as pltpu
from jax.experimental.pallas import tpu_sc as plsc

VOCAB = 3
T_DIM = 4
NUM_CORES = 2
NUM_SUBCORES = 16
LANES = 16
NUM_WORKERS = NUM_CORES * NUM_SUBCORES
UNITS_PER_TILE = 3
UNIT = 2048


def _make_sc_gather(n_idx: int, tbl_len: int):
    n_out = n_idx * VOCAB
    col_len = n_idx // T_DIM
    blocks = col_len // UNIT
    groups = UNIT // LANES

    mesh = plsc.VectorSubcoreMesh(core_axis_name="c", subcore_axis_name="s")

    @functools.partial(
        pl.kernel,
        out_type=jax.ShapeDtypeStruct((n_out,), jnp.float32),
        mesh=mesh,
        scratch_types=[
            pltpu.VMEM((UNIT,), jnp.int32),
            pltpu.VMEM((tbl_len,), jnp.float32),
            pltpu.VMEM((UNIT,), jnp.float32),
        ],
        compiler_params=pltpu.CompilerParams(needs_layout_passes=False),
    )
    def sc_gather(idx_hbm, tbl_hbm, out_hbm, idx_v, tbl_v, out_v):
        wid = lax.axis_index("s") * NUM_CORES + lax.axis_index("c")
        pltpu.sync_copy(tbl_hbm, tbl_v)

        for r in range(UNITS_PER_TILE):
            u = wid * UNITS_PER_TILE + r
            s = u // blocks
            b = u % blocks
            j = s % T_DIM
            c = s // T_DIM
            scale = jnp.where(
                j == 0, 1, jnp.where(j == 1, 3, jnp.where(j == 2, 9, 27))
            )
            base_c = c * (VOCAB ** T_DIM)

            pltpu.sync_copy(idx_hbm.at[pl.ds(j * col_len + b * UNIT, UNIT)], idx_v)

            def body(g, carry):
                iv = idx_v[pl.ds(g * LANES, LANES)]
                vals = plsc.load_gather(tbl_v, [iv * scale + base_c])
                out_v[pl.ds(g * LANES, LANES)] = vals
                return carry

            lax.fori_loop(0, groups, body, 0)
            pltpu.sync_copy(out_v, out_hbm.at[pl.ds(s * col_len + b * UNIT, UNIT)])

    return sc_gather


def kernel(idx, outputs):
    b, t = idx.shape
    idx_cols = idx.T.reshape(-1).astype(jnp.int32)
    tbl_cols = outputs.T.reshape(-1)
    out_flat = _make_sc_gather(b * t, tbl_cols.shape[0])(idx_cols, tbl_cols)
    return out_flat.reshape(VOCAB, t, b).transpose(2, 1, 0)


# speedup vs baseline: 3.6529x; 1.1659x over previous
import functools

import jax
import jax.numpy as jnp
from jax import lax
from jax.experimental import pallas as pl
from jax.experimental.pallas import tpu as pltpu
from jax.experimental.pallas import tpu_sc as plsc

VOCAB = 3
T_DIM = 4
TBL_ROWS = VOCAB ** T_DIM
NUM_CORES = 2
NUM_SUBCORES = 16
LANES = 16
NUM_WORKERS = NUM_CORES * NUM_SUBCORES


def _make_sc_gather(n_idx: int, tbl_len: int):
    chunk = n_idx // NUM_WORKERS
    groups = chunk // LANES

    mesh = plsc.VectorSubcoreMesh(core_axis_name="c", subcore_axis_name="s")

    @functools.partial(
        pl.kernel,
        out_type=jax.ShapeDtypeStruct((n_idx * VOCAB,), jnp.float32),
        mesh=mesh,
        scratch_types=[
            pltpu.VMEM((chunk,), jnp.int32),
            pltpu.VMEM((tbl_len,), jnp.float32),
            pltpu.VMEM((chunk * VOCAB,), jnp.float32),
        ],
        compiler_params=pltpu.CompilerParams(needs_layout_passes=False),
    )
    def sc_gather(idx_hbm, tbl_hbm, out_hbm, idx_v, tbl_v, out_v):
        wid = lax.axis_index("s") * NUM_CORES + lax.axis_index("c")
        base = wid * chunk
        pltpu.sync_copy(idx_hbm.at[pl.ds(base, chunk)], idx_v)
        pltpu.sync_copy(tbl_hbm, tbl_v)

        def body(g, carry):
            jg = lax.shift_right_logical(g, 3) & 3
            scale = jnp.where(
                jg == 0, 1, jnp.where(jg == 1, 3, jnp.where(jg == 2, 9, 27))
            )
            iv = idx_v[pl.ds(g * LANES, LANES)]
            f = iv * scale
            for c in range(VOCAB):
                vals = plsc.load_gather(tbl_v, [f + c * TBL_ROWS])
                out_v[pl.ds(c * chunk + g * LANES, LANES)] = vals
            return carry

        lax.fori_loop(0, groups, body, 0)
        for c in range(VOCAB):
            pltpu.sync_copy(
                out_v.at[pl.ds(c * chunk, chunk)],
                out_hbm.at[pl.ds(c * n_idx + base, chunk)],
            )

    return sc_gather


def kernel(idx, outputs):
    b, t = idx.shape
    ihi = b // 128
    idx_nat = (
        idx.reshape(ihi, 128, t).transpose(0, 2, 1).reshape(-1).astype(jnp.int32)
    )
    tbl_cols = outputs.T.reshape(-1)
    out_flat = _make_sc_gather(b * t, tbl_cols.shape[0])(idx_nat, tbl_cols)
    return (
        out_flat.reshape(VOCAB, ihi, t, 128)
        .transpose(1, 3, 2, 0)
        .reshape(b, t, VOCAB)
    )


# speedup vs baseline: 4.0010x; 1.0953x over previous
import functools

import jax
import jax.numpy as jnp
from jax import lax
from jax.experimental import pallas as pl
from jax.experimental.pallas import tpu as pltpu
from jax.experimental.pallas import tpu_sc as plsc

VOCAB = 3
T_DIM = 4
TBL_ROWS = VOCAB ** T_DIM
NUM_CORES = 2
NUM_SUBCORES = 16
LANES = 16
NUM_WORKERS = NUM_CORES * NUM_SUBCORES


def _make_sc_gather(n_idx: int, tbl_len: int):
    chunk = n_idx // NUM_WORKERS
    groups = chunk // LANES

    mesh = plsc.VectorSubcoreMesh(core_axis_name="c", subcore_axis_name="s")

    @functools.partial(
        pl.kernel,
        out_type=jax.ShapeDtypeStruct((n_idx * VOCAB,), jnp.float32),
        mesh=mesh,
        scratch_types=[
            pltpu.VMEM((chunk,), jnp.int32),
            pltpu.VMEM((tbl_len,), jnp.float32),
            pltpu.VMEM((chunk * VOCAB,), jnp.float32),
            pltpu.SemaphoreType.DMA,
            pltpu.SemaphoreType.DMA,
            pltpu.SemaphoreType.DMA,
        ],
        compiler_params=pltpu.CompilerParams(needs_layout_passes=False),
    )
    def sc_gather(idx_hbm, tbl_hbm, out_hbm, idx_v, tbl_v, out_v, s0, s1, s2):
        wid = lax.axis_index("s") * NUM_CORES + lax.axis_index("c")
        base = wid * chunk
        cp_idx = pltpu.async_copy(idx_hbm.at[pl.ds(base, chunk)], idx_v, s0)
        cp_tbl = pltpu.async_copy(tbl_hbm, tbl_v, s1)
        cp_idx.wait()
        cp_tbl.wait()

        @plsc.parallel_loop(0, groups, unroll=4)
        def body(g):
            jg = lax.shift_right_logical(g, 3) & 3
            scale = jnp.where(
                jg == 0, 1, jnp.where(jg == 1, 3, jnp.where(jg == 2, 9, 27))
            )
            iv = idx_v[pl.ds(g * LANES, LANES)]
            f = iv * scale
            for c in range(VOCAB):
                vals = plsc.load_gather(tbl_v, [f + c * TBL_ROWS])
                out_v[pl.ds(c * chunk + g * LANES, LANES)] = vals

        sems = (s0, s1, s2)
        cps = [
            pltpu.async_copy(
                out_v.at[pl.ds(c * chunk, chunk)],
                out_hbm.at[pl.ds(c * n_idx + base, chunk)],
                sems[c],
            )
            for c in range(VOCAB)
        ]
        for cp in cps:
            cp.wait()

    return sc_gather


def kernel(idx, outputs):
    b, t = idx.shape
    ihi = b // 128
    idx_nat = (
        idx.reshape(ihi, 128, t).transpose(0, 2, 1).reshape(-1).astype(jnp.int32)
    )
    tbl_cols = outputs.T.reshape(-1)
    out_flat = _make_sc_gather(b * t, tbl_cols.shape[0])(idx_nat, tbl_cols)
    return (
        out_flat.reshape(VOCAB, ihi, t, 128)
        .transpose(1, 3, 2, 0)
        .reshape(b, t, VOCAB)
    )


# speedup vs baseline: 4.3523x; 1.0878x over previous
import functools

import jax
import jax.numpy as jnp
from jax import lax
from jax.experimental import pallas as pl
from jax.experimental.pallas import tpu as pltpu
from jax.experimental.pallas import tpu_sc as plsc

VOCAB = 3
T_DIM = 4
TBL_ROWS = VOCAB ** T_DIM
NUM_CORES = 1
NUM_SUBCORES = 16
LANES = 16
NUM_WORKERS = NUM_CORES * NUM_SUBCORES


def _make_sc_gather(n_idx: int, tbl_len: int):
    chunk = n_idx // NUM_WORKERS
    groups = chunk // LANES

    mesh = plsc.VectorSubcoreMesh(
        core_axis_name="c", subcore_axis_name="s", num_cores=NUM_CORES
    )

    @functools.partial(
        pl.kernel,
        out_type=jax.ShapeDtypeStruct((n_idx * VOCAB,), jnp.float32),
        mesh=mesh,
        scratch_types=[
            pltpu.VMEM((chunk,), jnp.int32),
            pltpu.VMEM((tbl_len,), jnp.float32),
            pltpu.VMEM((chunk * VOCAB,), jnp.float32),
            pltpu.SemaphoreType.DMA,
            pltpu.SemaphoreType.DMA,
            pltpu.SemaphoreType.DMA,
        ],
        compiler_params=pltpu.CompilerParams(needs_layout_passes=False),
    )
    def sc_gather(idx_hbm, tbl_hbm, out_hbm, idx_v, tbl_v, out_v, s0, s1, s2):
        wid = lax.axis_index("s") * NUM_CORES + lax.axis_index("c")
        base = wid * chunk
        cp_idx = pltpu.async_copy(idx_hbm.at[pl.ds(base, chunk)], idx_v, s0)
        cp_tbl = pltpu.async_copy(tbl_hbm, tbl_v, s1)
        cp_idx.wait()
        cp_tbl.wait()

        @plsc.parallel_loop(0, groups, unroll=4)
        def body(g):
            jg = lax.shift_right_logical(g, 3) & 3
            scale = jnp.where(
                jg == 0, 1, jnp.where(jg == 1, 3, jnp.where(jg == 2, 9, 27))
            )
            iv = idx_v[pl.ds(g * LANES, LANES)]
            f = iv * scale
            for c in range(VOCAB):
                vals = plsc.load_gather(tbl_v, [f + c * TBL_ROWS])
                out_v[pl.ds(c * chunk + g * LANES, LANES)] = vals

        sems = (s0, s1, s2)
        cps = [
            pltpu.async_copy(
                out_v.at[pl.ds(c * chunk, chunk)],
                out_hbm.at[pl.ds(c * n_idx + base, chunk)],
                sems[c],
            )
            for c in range(VOCAB)
        ]
        for cp in cps:
            cp.wait()

    return sc_gather


def kernel(idx, outputs):
    b, t = idx.shape
    ihi = b // 128
    idx_nat = (
        idx.reshape(ihi, 128, t).transpose(0, 2, 1).reshape(-1).astype(jnp.int32)
    )
    tbl_cols = outputs.T.reshape(-1)
    out_flat = _make_sc_gather(b * t, tbl_cols.shape[0])(idx_nat, tbl_cols)
    return (
        out_flat.reshape(VOCAB, ihi, t, 128)
        .transpose(1, 3, 2, 0)
        .reshape(b, t, VOCAB)
    )
